# baseline (speedup 1.0000x reference)
import functools

import jax
import jax.numpy as jnp
from jax import lax
from jax.experimental import pallas as pl
from jax.experimental.pallas import tpu as pltpu
from jax.experimental.pallas import tpu_sc as plsc

N_NODES = 10000
N_EDGES = 320000
D_FEAT = 128
N_REL = 10

N_TILES = 16
DH = D_FEAT // 2
W = 512
IG = 64
NG = W // IG
N_CHUNKS = N_EDGES // W
CPT = N_CHUNKS // N_TILES
ROWS_PER_TILE = 624


def _sc_body(emb2, tail, head2, etype, aug, w2, out, acc,
             tail0, et0, aug0, gidx0, head0, rows0,
             tail1, et1, aug1, gidx1, head1, rows1,
             w_v, sem_i, sem_g0, sem_g1, sem_s0, sem_s1):
    c = lax.axis_index("c")
    s = lax.axis_index("s")
    chunk0 = s * CPT

    B0 = (tail0, et0, aug0, gidx0, head0, rows0, sem_g0, sem_s0)
    B1 = (tail1, et1, aug1, gidx1, head1, rows1, sem_g1, sem_s1)

    def idx_load(B, ch):
        tl, et, ag, gx, hd, rw, sg, ss = B
        base = ch * W
        pltpu.async_copy(tail.at[pl.ds(base, W)], tl, sem_i)
        pltpu.async_copy(etype.at[pl.ds(base, W)], et, sem_i)
        pltpu.async_copy(aug.at[pl.ds(base, W)], ag, sem_i)
        pltpu.async_copy(head2.at[pl.ds(ch * NG, NG)], hd, sem_i)
        pltpu.make_async_copy(tail.at[pl.ds(base, W)], tl, sem_i).wait()
        pltpu.make_async_copy(etype.at[pl.ds(base, W)], et, sem_i).wait()
        pltpu.make_async_copy(aug.at[pl.ds(base, W)], ag, sem_i).wait()
        pltpu.make_async_copy(head2.at[pl.ds(ch * NG, NG)], hd, sem_i).wait()

    def gidx_compute(B):
        tl, et, ag, gx, hd, rw, sg, ss = B

        def gi(i, _):
            for k in range(IG // 16):
                t = tl[pl.ds(i * IG + k * 16, 16)]
                gx[i, pl.ds(k * 16, 16)] = t * 2 + c
            return 0

        lax.fori_loop(0, NG, gi, 0)

    def gather_start(B):
        tl, et, ag, gx, hd, rw, sg, ss = B
        for j in range(NG):
            pltpu.async_copy(emb2.at[gx.at[j]], rw.at[pl.ds(j * IG, IG)], sg)

    def gather_wait(B):
        tl, et, ag, gx, hd, rw, sg, ss = B
        for j in range(NG):
            pltpu.make_async_copy(emb2.at[gx.at[j]],
                                  rw.at[pl.ds(j * IG, IG)], sg).wait()

    def compute(B):
        tl, et, ag, gx, hd, rw, sg, ss = B
        nk = DH // 16

        def ce(g16, _):
            et16 = et[pl.ds(g16 * 16, 16)]
            a16 = ag[pl.ds(g16 * 16, 16)]
            wb16 = et16 * 2 + c
            for l in range(16):
                wb = wb16[l]
                a = a16[l]
                e = g16 * 16 + l
                eks = [rw[e, pl.ds(k * 16, 16)] for k in range(nk)]
                wks = [w_v[wb, pl.ds(k * 16, 16)] for k in range(nk)]
                ps = [eks[k] * wks[k] * a for k in range(nk)]
                for k in range(nk):
                    rw[e, pl.ds(k * 16, 16)] = ps[k]
            return 0

        lax.fori_loop(0, W // 16, ce, 0)

    def scatter_start(B):
        tl, et, ag, gx, hd, rw, sg, ss = B
        for j in range(NG):
            pltpu.async_copy(rw.at[pl.ds(j * IG, IG)], acc.at[hd.at[j]], ss,
                             add=True)

    def scatter_drain(B):
        tl, et, ag, gx, hd, rw, sg, ss = B
        for j in range(NG):
            pltpu.make_async_copy(rw.at[pl.ds(j * IG, IG)],
                                  acc.at[hd.at[j]], ss).wait()

    zeros16 = jnp.zeros((16,), jnp.float32)

    def zrow(i, _):
        for k in range(DH // 16):
            rows0[i, pl.ds(k * 16, 16)] = zeros16
        return 0

    lax.fori_loop(0, W, zrow, 0)
    r0 = s * ROWS_PER_TILE
    pltpu.sync_copy(rows0, acc.at[pl.ds(r0, W)])
    pltpu.sync_copy(rows0.at[pl.ds(0, ROWS_PER_TILE - W)],
                    acc.at[pl.ds(r0 + W, ROWS_PER_TILE - W)])

    @pl.when(s == N_TILES - 1)
    def _():
        pltpu.sync_copy(rows0.at[pl.ds(0, N_NODES - N_TILES * ROWS_PER_TILE)],
                        acc.at[pl.ds(N_TILES * ROWS_PER_TILE,
                                     N_NODES - N_TILES * ROWS_PER_TILE)])

    pltpu.sync_copy(w2, w_v)
    plsc.subcore_barrier()

    idx_load(B0, chunk0)
    gidx_compute(B0)
    gather_start(B0)
    idx_load(B1, chunk0 + 1)
    gidx_compute(B1)
    gather_start(B1)
    gather_wait(B0)
    compute(B0)
    scatter_start(B0)

    def pair(i, _):
        gather_wait(B1)
        scatter_drain(B0)
        idx_load(B0, chunk0 + 2 * i + 2)
        gidx_compute(B0)
        gather_start(B0)
        compute(B1)
        scatter_start(B1)
        gather_wait(B0)
        scatter_drain(B1)
        idx_load(B1, chunk0 + 2 * i + 3)
        gidx_compute(B1)
        gather_start(B1)
        compute(B0)
        scatter_start(B0)
        return 0

    lax.fori_loop(0, (CPT - 3) // 2, pair, 0)

    gather_wait(B1)
    scatter_drain(B0)
    idx_load(B0, chunk0 + CPT - 1)
    gidx_compute(B0)
    gather_start(B0)
    compute(B1)
    scatter_start(B1)
    gather_wait(B0)
    scatter_drain(B1)
    compute(B0)
    scatter_start(B0)
    scatter_drain(B0)

    @pl.when(s == N_TILES - 1)
    def _():
        idx_load(B1, N_CHUNKS - 1)
        gidx_compute(B1)
        gather_start(B1)
        gather_wait(B1)
        compute(B1)
        scatter_start(B1)
        scatter_drain(B1)

    plsc.subcore_barrier()

    pltpu.sync_copy(acc.at[pl.ds(r0, ROWS_PER_TILE)],
                    out.at[c, pl.ds(r0, ROWS_PER_TILE), :])

    @pl.when(s == N_TILES - 1)
    def _():
        tail_rows = N_NODES - N_TILES * ROWS_PER_TILE
        pltpu.sync_copy(acc.at[pl.ds(N_TILES * ROWS_PER_TILE, tail_rows)],
                        out.at[c, pl.ds(N_TILES * ROWS_PER_TILE, tail_rows), :])


def kernel(all_emb, edge_index, edge_type, weight, aug_edge_weight):
    emb2 = all_emb.reshape(2 * N_NODES, DH)
    tail = edge_index[1].astype(jnp.int32)
    head2 = edge_index[0].astype(jnp.int32).reshape(N_EDGES // IG, IG)
    etype = edge_type.astype(jnp.int32)
    aug = aug_edge_weight.reshape(N_EDGES)
    w2 = weight.reshape(2 * N_REL, DH)

    mesh = plsc.VectorSubcoreMesh(core_axis_name="c", subcore_axis_name="s")
    buf = lambda: [
        pltpu.VMEM((W,), jnp.int32),
        pltpu.VMEM((W,), jnp.int32),
        pltpu.VMEM((W,), jnp.float32),
        pltpu.VMEM((NG, IG), jnp.int32),
        pltpu.VMEM((NG, IG), jnp.int32),
        pltpu.VMEM((W, DH), jnp.float32),
    ]
    f = functools.partial(
        pl.kernel,
        mesh=mesh,
        compiler_params=pltpu.CompilerParams(use_tc_tiling_on_sc=False),
        out_type=jax.ShapeDtypeStruct((2, N_NODES, DH), jnp.float32),
        scratch_types=[
            pltpu.VMEM_SHARED((N_NODES, DH), jnp.float32),
            *buf(), *buf(),
            pltpu.VMEM((2 * N_REL, DH), jnp.float32),
            pltpu.SemaphoreType.DMA,
            pltpu.SemaphoreType.DMA,
            pltpu.SemaphoreType.DMA,
            pltpu.SemaphoreType.DMA,
            pltpu.SemaphoreType.DMA,
        ],
    )(_sc_body)
    halves = f(emb2, tail, head2, etype, aug, w2)
    return jnp.concatenate([halves[0], halves[1]], axis=1)


# speedup vs baseline: 1.2723x; 1.2723x over previous
import functools

import jax
import jax.numpy as jnp
from jax import lax
from jax.experimental import pallas as pl
from jax.experimental.pallas import tpu as pltpu
from jax.experimental.pallas import tpu_sc as plsc

N_NODES = 10000
N_EDGES = 320000
D_FEAT = 128
N_REL = 10

N_TILES = 16
DH = D_FEAT // 2
W = 512
IG = 64
NG = W // IG
N_CHUNKS = N_EDGES // W
CPT = N_CHUNKS // N_TILES
ROWS_PER_TILE = 624


def _sc_body(emb2, tail, head2, etype, aug, w2, out, acc,
             tail0, et0, aug0, gidx0, head0, rows0,
             tail1, et1, aug1, gidx1, head1, rows1,
             w_v, sem_i, sem_g0, sem_g1, sem_s0, sem_s1):
    c = lax.axis_index("c")
    s = lax.axis_index("s")
    chunk0 = s * CPT

    B0 = (tail0, et0, aug0, gidx0, head0, rows0, sem_g0, sem_s0)
    B1 = (tail1, et1, aug1, gidx1, head1, rows1, sem_g1, sem_s1)

    def idx_load(B, ch):
        tl, et, ag, gx, hd, rw, sg, ss = B
        base = ch * W
        pltpu.async_copy(tail.at[pl.ds(base, W)], tl, sem_i)
        pltpu.async_copy(etype.at[pl.ds(base, W)], et, sem_i)
        pltpu.async_copy(aug.at[pl.ds(base, W)], ag, sem_i)
        pltpu.async_copy(head2.at[pl.ds(ch * NG, NG)], hd, sem_i)
        pltpu.make_async_copy(tail.at[pl.ds(base, W)], tl, sem_i).wait()
        pltpu.make_async_copy(etype.at[pl.ds(base, W)], et, sem_i).wait()
        pltpu.make_async_copy(aug.at[pl.ds(base, W)], ag, sem_i).wait()
        pltpu.make_async_copy(head2.at[pl.ds(ch * NG, NG)], hd, sem_i).wait()

    def gidx_compute(B):
        tl, et, ag, gx, hd, rw, sg, ss = B

        def gi(i, _):
            for k in range(IG // 16):
                t = tl[pl.ds(i * IG + k * 16, 16)]
                gx[i, pl.ds(k * 16, 16)] = t * 2 + c
            return 0

        lax.fori_loop(0, NG, gi, 0)

    def gather_start(B):
        tl, et, ag, gx, hd, rw, sg, ss = B
        for j in range(NG):
            pltpu.async_copy(emb2.at[gx.at[j]], rw.at[pl.ds(j * IG, IG)], sg)

    def gather_wait(B):
        tl, et, ag, gx, hd, rw, sg, ss = B
        for j in range(NG):
            pltpu.make_async_copy(emb2.at[gx.at[j]],
                                  rw.at[pl.ds(j * IG, IG)], sg).wait()

    def compute(B):
        tl, et, ag, gx, hd, rw, sg, ss = B
        nk = DH // 16

        def ce(g16, _):
            et16 = et[pl.ds(g16 * 16, 16)]
            a16 = ag[pl.ds(g16 * 16, 16)]
            wb16 = et16 * 2 + c
            for l in range(0, 16, 2):
                ea = g16 * 16 + l
                eb = ea + 1
                wba = wb16[l]
                wbb = wb16[l + 1]
                aa = a16[l]
                ab = a16[l + 1]
                eksa = [rw[ea, pl.ds(k * 16, 16)] for k in range(nk)]
                wksa = [w_v[wba, pl.ds(k * 16, 16)] for k in range(nk)]
                eksb = [rw[eb, pl.ds(k * 16, 16)] for k in range(nk)]
                wksb = [w_v[wbb, pl.ds(k * 16, 16)] for k in range(nk)]
                psa = [eksa[k] * wksa[k] * aa for k in range(nk)]
                psb = [eksb[k] * wksb[k] * ab for k in range(nk)]
                for k in range(nk):
                    rw[ea, pl.ds(k * 16, 16)] = psa[k]
                for k in range(nk):
                    rw[eb, pl.ds(k * 16, 16)] = psb[k]
            return 0

        lax.fori_loop(0, W // 16, ce, 0)

    def scatter_start(B):
        tl, et, ag, gx, hd, rw, sg, ss = B
        for j in range(NG):
            pltpu.async_copy(rw.at[pl.ds(j * IG, IG)], acc.at[hd.at[j]], ss,
                             add=True)

    def scatter_drain(B):
        tl, et, ag, gx, hd, rw, sg, ss = B
        for j in range(NG):
            pltpu.make_async_copy(rw.at[pl.ds(j * IG, IG)],
                                  acc.at[hd.at[j]], ss).wait()

    zeros16 = jnp.zeros((16,), jnp.float32)

    def zrow(i, _):
        for k in range(DH // 16):
            rows0[i, pl.ds(k * 16, 16)] = zeros16
        return 0

    lax.fori_loop(0, W, zrow, 0)
    r0 = s * ROWS_PER_TILE
    pltpu.sync_copy(rows0, acc.at[pl.ds(r0, W)])
    pltpu.sync_copy(rows0.at[pl.ds(0, ROWS_PER_TILE - W)],
                    acc.at[pl.ds(r0 + W, ROWS_PER_TILE - W)])

    @pl.when(s == N_TILES - 1)
    def _():
        pltpu.sync_copy(rows0.at[pl.ds(0, N_NODES - N_TILES * ROWS_PER_TILE)],
                        acc.at[pl.ds(N_TILES * ROWS_PER_TILE,
                                     N_NODES - N_TILES * ROWS_PER_TILE)])

    pltpu.sync_copy(w2, w_v)
    plsc.subcore_barrier()

    idx_load(B0, chunk0)
    gidx_compute(B0)
    gather_start(B0)
    idx_load(B1, chunk0 + 1)
    gidx_compute(B1)
    gather_start(B1)
    gather_wait(B0)
    compute(B0)
    scatter_start(B0)

    def pair(i, _):
        gather_wait(B1)
        scatter_drain(B0)
        idx_load(B0, chunk0 + 2 * i + 2)
        gidx_compute(B0)
        gather_start(B0)
        compute(B1)
        scatter_start(B1)
        gather_wait(B0)
        scatter_drain(B1)
        idx_load(B1, chunk0 + 2 * i + 3)
        gidx_compute(B1)
        gather_start(B1)
        compute(B0)
        scatter_start(B0)
        return 0

    lax.fori_loop(0, (CPT - 3) // 2, pair, 0)

    gather_wait(B1)
    scatter_drain(B0)
    idx_load(B0, chunk0 + CPT - 1)
    gidx_compute(B0)
    gather_start(B0)
    compute(B1)
    scatter_start(B1)
    gather_wait(B0)
    scatter_drain(B1)
    compute(B0)
    scatter_start(B0)
    scatter_drain(B0)

    @pl.when(s == N_TILES - 1)
    def _():
        idx_load(B1, N_CHUNKS - 1)
        gidx_compute(B1)
        gather_start(B1)
        gather_wait(B1)
        compute(B1)
        scatter_start(B1)
        scatter_drain(B1)

    plsc.subcore_barrier()

    pltpu.sync_copy(acc.at[pl.ds(r0, ROWS_PER_TILE)],
                    out.at[c, pl.ds(r0, ROWS_PER_TILE), :])

    @pl.when(s == N_TILES - 1)
    def _():
        tail_rows = N_NODES - N_TILES * ROWS_PER_TILE
        pltpu.sync_copy(acc.at[pl.ds(N_TILES * ROWS_PER_TILE, tail_rows)],
                        out.at[c, pl.ds(N_TILES * ROWS_PER_TILE, tail_rows), :])


def kernel(all_emb, edge_index, edge_type, weight, aug_edge_weight):
    emb2 = all_emb.reshape(2 * N_NODES, DH)
    tail = edge_index[1].astype(jnp.int32)
    head2 = edge_index[0].astype(jnp.int32).reshape(N_EDGES // IG, IG)
    etype = edge_type.astype(jnp.int32)
    aug = aug_edge_weight.reshape(N_EDGES)
    w2 = weight.reshape(2 * N_REL, DH)

    mesh = plsc.VectorSubcoreMesh(core_axis_name="c", subcore_axis_name="s")
    buf = lambda: [
        pltpu.VMEM((W,), jnp.int32),
        pltpu.VMEM((W,), jnp.int32),
        pltpu.VMEM((W,), jnp.float32),
        pltpu.VMEM((NG, IG), jnp.int32),
        pltpu.VMEM((NG, IG), jnp.int32),
        pltpu.VMEM((W, DH), jnp.float32),
    ]
    f = functools.partial(
        pl.kernel,
        mesh=mesh,
        compiler_params=pltpu.CompilerParams(use_tc_tiling_on_sc=False),
        out_type=jax.ShapeDtypeStruct((2, N_NODES, DH), jnp.float32),
        scratch_types=[
            pltpu.VMEM_SHARED((N_NODES, DH), jnp.float32),
            *buf(), *buf(),
            pltpu.VMEM((2 * N_REL, DH), jnp.float32),
            pltpu.SemaphoreType.DMA,
            pltpu.SemaphoreType.DMA,
            pltpu.SemaphoreType.DMA,
            pltpu.SemaphoreType.DMA,
            pltpu.SemaphoreType.DMA,
        ],
    )(_sc_body)
    halves = f(emb2, tail, head2, etype, aug, w2)
    return jnp.concatenate([halves[0], halves[1]], axis=1)
